# Initial kernel scaffold; baseline (speedup 1.0000x reference)
import jax
import jax.numpy as jnp
from jax.experimental import pallas as pl


def kernel(x, W_proj, b_proj, W_g1, b_g1, W_g2, b_g2, W_cls, b_cls):
    raise NotImplementedError("write your pallas kernel here")


# speedup vs baseline: 20.2036x; 20.2036x over previous
import functools

import jax
import jax.numpy as jnp
from jax import lax
from jax.experimental import pallas as pl
from jax.experimental.pallas import tpu as pltpu
from jax.experimental.pallas import tpu_sc as plsc

N = 10000
NP = 10240
D = 512
H = 128
K = 5
RT = 256
CT = 1280
NRT = NP // RT
NCT = NP // CT
INF = float("inf")
BIG = 2**30

_NC = 2
_NS = 16
_NW = _NC * _NS
_SLAB = NP // _NW
_CH = 64
_NCHK = _SLAB // _CH
_RPS = NP // _NS

_HIGH = jax.lax.Precision.HIGHEST


def _proj_body(x_ref, w_ref, b_ref, h_ref, sq_ref, sqt_ref):
    i = pl.program_id(0)
    h = jnp.dot(x_ref[...], w_ref[...], preferred_element_type=jnp.float32,
                precision=_HIGH)
    h = jnp.maximum(h + b_ref[...], 0.0)
    row = i * RT + lax.broadcasted_iota(jnp.int32, (RT, 1), 0)
    valid = row < N
    h = jnp.where(valid, h, 0.0)
    sq = jnp.sum(h * h, axis=1, keepdims=True)
    sq = jnp.where(valid, sq, 1e30)
    h_ref[...] = h
    sq_ref[...] = sq
    sqt_ref[...] = sq.reshape(1, RT)


def _proj(x_pad, W_proj, b_proj):
    return pl.pallas_call(
        _proj_body,
        grid=(NRT,),
        in_specs=[
            pl.BlockSpec((RT, D), lambda i: (i, 0)),
            pl.BlockSpec((D, H), lambda i: (0, 0)),
            pl.BlockSpec((1, H), lambda i: (0, 0)),
        ],
        out_specs=[
            pl.BlockSpec((RT, H), lambda i: (i, 0)),
            pl.BlockSpec((RT, 1), lambda i: (i, 0)),
            pl.BlockSpec((1, RT), lambda i: (0, i)),
        ],
        out_shape=[
            jax.ShapeDtypeStruct((NP, H), jnp.float32),
            jax.ShapeDtypeStruct((NP, 1), jnp.float32),
            jax.ShapeDtypeStruct((1, NP), jnp.float32),
        ],
    )(x_pad, W_proj, b_proj)


def _extract6(v, gi, width):
    pv = jnp.full((RT, 8), INF, jnp.float32)
    pi = jnp.full((RT, 8), BIG, jnp.int32)
    lane8 = lax.broadcasted_iota(jnp.int32, (RT, 8), 1)
    for p in range(K + 1):
        m = jnp.min(v, axis=1, keepdims=True)
        sel = jnp.min(jnp.where(v == m, gi, BIG), axis=1, keepdims=True)
        pv = jnp.where(lane8 == p, m, pv)
        pi = jnp.where(lane8 == p, sel, pi)
        v = jnp.where(gi == sel, INF, v)
    return pv, pi


def _knn_body(hr_ref, sqr_ref, hc_ref, sqct_ref, out_ref, runv_ref, runi_ref):
    j = pl.program_id(1)

    @pl.when(j == 0)
    def _init():
        runv_ref[...] = jnp.full((RT, 8), INF, jnp.float32)
        runi_ref[...] = jnp.full((RT, 8), BIG, jnp.int32)

    d2 = (sqr_ref[...] + sqct_ref[...]
          - 2.0 * jnp.dot(hr_ref[...], hc_ref[...].T,
                          preferred_element_type=jnp.float32,
                          precision=_HIGH))
    dist = jnp.sqrt(jnp.maximum(d2, 0.0))
    gi = j * CT + lax.broadcasted_iota(jnp.int32, (RT, CT), 1)
    tv, ti = _extract6(dist, gi, CT)

    cat_v = jnp.concatenate([runv_ref[...], tv], axis=1)
    cat_i = jnp.concatenate([runi_ref[...], ti], axis=1)
    mv, mi = _extract6(cat_v, cat_i, 16)
    runv_ref[...] = mv
    runi_ref[...] = mi

    @pl.when(j == NCT - 1)
    def _emit():
        out_ref[...] = mi


def _knn(h_pad, sq, sqT):
    return pl.pallas_call(
        _knn_body,
        grid=(NRT, NCT),
        in_specs=[
            pl.BlockSpec((RT, H), lambda i, j: (i, 0)),
            pl.BlockSpec((RT, 1), lambda i, j: (i, 0)),
            pl.BlockSpec((CT, H), lambda i, j: (j, 0)),
            pl.BlockSpec((1, CT), lambda i, j: (0, j)),
        ],
        out_specs=pl.BlockSpec((RT, 8), lambda i, j: (i, 0)),
        out_shape=jax.ShapeDtypeStruct((NP, 8), jnp.int32),
        scratch_shapes=[
            pltpu.VMEM((RT, 8), jnp.float32),
            pltpu.VMEM((RT, 8), jnp.int32),
        ],
    )(h_pad, sq, h_pad, sqT)


@functools.cache
def _mesh():
    return plsc.VectorSubcoreMesh(core_axis_name="c", subcore_axis_name="s",
                                  num_cores=_NC, num_subcores=_NS)


@functools.cache
def _indeg_sc():
    return pl.kernel(
        _indeg_body,
        out_type=jax.ShapeDtypeStruct((_NW * NP,), jnp.float32),
        mesh=_mesh(),
        scratch_types=[
            pltpu.VMEM((NP,), jnp.float32),
            pltpu.VMEM((K * _SLAB,), jnp.int32),
        ],
        compiler_params=pltpu.CompilerParams(needs_layout_passes=False),
    )


def _indeg_body(idxk_hbm, out_hbm, hist, ibuf):
    c = lax.axis_index("c")
    s = lax.axis_index("s")
    wid = s * _NC + c
    base = wid * _SLAB

    for k in range(K):
        pltpu.sync_copy(idxk_hbm.at[pl.ds(k * NP + base, _SLAB)],
                        ibuf.at[pl.ds(k * _SLAB, _SLAB)])

    def _zero(i, carry):
        hist[pl.ds(i * 16, 16)] = jnp.zeros((16,), jnp.float32)
        return carry
    lax.fori_loop(0, NP // 16, _zero, 0)

    ones = jnp.ones((16,), jnp.float32)
    lane = lax.broadcasted_iota(jnp.int32, (16,), 0)
    for k in range(K):
        def _acc(i, carry, k=k):
            iv = ibuf[pl.ds(k * _SLAB + i * 16, 16)]
            src = base + i * 16 + lane
            iv = jnp.where(src < N, iv, N)
            plsc.addupdate_scatter(hist, [iv], ones)
            return carry
        lax.fori_loop(0, _SLAB // 16, _acc, 0)

    pltpu.sync_copy(hist, out_hbm.at[pl.ds(wid * NP, NP)])


_HALF = NP // 2
_HPS = _HALF // _NS


@functools.cache
def _mp_sc():
    return pl.kernel(
        _mp_body,
        out_type=(
            jax.ShapeDtypeStruct((NP, H), jnp.float32),
            jax.ShapeDtypeStruct((_NC, NP, H), jnp.float32),
        ),
        mesh=_mesh(),
        scratch_types=[
            pltpu.VMEM((K * _SLAB,), jnp.int32),
            pltpu.VMEM((K, _CH, H), jnp.float32),
            pltpu.VMEM((_CH, H), jnp.float32),
            pltpu.VMEM((_CH, H), jnp.float32),
            pltpu.VMEM_SHARED((_HALF + 8, H), jnp.float32),
            pltpu.SemaphoreType.DMA,
            pltpu.SemaphoreType.DMA,
        ],
        compiler_params=pltpu.CompilerParams(needs_layout_passes=False),
    )


def _mp_body(y_hbm, idxk_hbm, outg_hbm, outs_hbm, ibuf, gb, yb, zb,
             spacc, sem, sem2):
    c = lax.axis_index("c")
    s = lax.axis_index("s")
    wid = s * _NC + c
    base = wid * _SLAB
    nq = _CH // 16

    zeros16 = jnp.zeros((16,), jnp.float32)

    def _zrow(r, carry):
        for l in range(H // 16):
            zb[r, pl.ds(l * 16, 16)] = zeros16
        return carry
    lax.fori_loop(0, _CH, _zrow, 0)

    for k in range(K):
        pltpu.sync_copy(idxk_hbm.at[pl.ds(k * NP + base, _SLAB)],
                        ibuf.at[pl.ds(k * _SLAB, _SLAB)])

    def _gchunk(ch, carry):
        cb = base + ch * _CH
        cps = []
        for k in range(K):
            for q in range(nq):
                iv = ibuf[pl.ds(k * _SLAB + ch * _CH + q * 16, 16)]
                cps.append(pltpu.async_copy(
                    y_hbm.at[iv], gb.at[k, pl.ds(q * 16, 16)], sem))
        for cp in cps:
            cp.wait()

        def _sum(r, c2):
            for l in range(H // 16):
                sl = pl.ds(l * 16, 16)
                acc = gb[0, r, sl]
                for k in range(1, K):
                    acc = acc + gb[k, r, sl]
                gb[0, r, sl] = acc
            return c2
        lax.fori_loop(0, _CH, _sum, 0)
        pltpu.sync_copy(gb.at[0], outg_hbm.at[pl.ds(cb, _CH)])
        return carry
    lax.fori_loop(0, _NCHK, _gchunk, 0)

    for half in range(2):
        lo = half * _HALF

        for t in range(_HPS // _CH):
            pltpu.sync_copy(zb, spacc.at[pl.ds(s * _HPS + t * _CH, _CH)])
        plsc.subcore_barrier()

        def _schunk(ch, carry, lo=lo):
            cb = base + ch * _CH
            pltpu.sync_copy(y_hbm.at[pl.ds(cb, _CH)], yb)
            cps = []
            for k in range(K):
                for q in range(nq):
                    iv = ibuf[pl.ds(k * _SLAB + ch * _CH + q * 16, 16)]
                    lv = iv - lo
                    ok = jnp.logical_and(lv >= 0, lv < _HALF)
                    lv = jnp.where(ok, lv, _HALF)
                    cps.append(pltpu.async_copy(
                        yb.at[pl.ds(q * 16, 16)], spacc.at[lv], sem2,
                        add=True))
            for cp in cps:
                cp.wait()
            return carry
        lax.fori_loop(0, _NCHK, _schunk, 0)

        plsc.subcore_barrier()
        pltpu.sync_copy(spacc.at[pl.ds(s * _HPS, _HPS)],
                        outs_hbm.at[c, pl.ds(lo + s * _HPS, _HPS)])
        plsc.subcore_barrier()


def _norm_body(pt_ref, h_ref, w_ref, dinv_ref, y_ref):
    deg = (K + 1.0) + jnp.sum(pt_ref[...], axis=1, keepdims=True)
    dinv = lax.rsqrt(deg)
    y = dinv * jnp.dot(h_ref[...], w_ref[...],
                       preferred_element_type=jnp.float32, precision=_HIGH)
    dinv_ref[...] = dinv
    y_ref[...] = y


def _norm(partsT, h_pad, W):
    return pl.pallas_call(
        _norm_body,
        grid=(NRT,),
        in_specs=[
            pl.BlockSpec((RT, _NW), lambda i: (i, 0)),
            pl.BlockSpec((RT, H), lambda i: (i, 0)),
            pl.BlockSpec((H, H), lambda i: (0, 0)),
        ],
        out_specs=[
            pl.BlockSpec((RT, 1), lambda i: (i, 0)),
            pl.BlockSpec((RT, H), lambda i: (i, 0)),
        ],
        out_shape=[
            jax.ShapeDtypeStruct((NP, 1), jnp.float32),
            jax.ShapeDtypeStruct((NP, H), jnp.float32),
        ],
    )(partsT, h_pad, W)


def _mid_body(og_ref, s0_ref, s1_ref, y1_ref, dinv_ref, b_ref, w_ref, y2_ref):
    i = pl.program_id(0)
    agg = og_ref[...] + s0_ref[...] + s1_ref[...] + y1_ref[...]
    g1 = jnp.maximum(dinv_ref[...] * agg + b_ref[...], 0.0)
    row = i * RT + lax.broadcasted_iota(jnp.int32, (RT, 1), 0)
    g1 = jnp.where(row < N, g1, 0.0)
    y2_ref[...] = dinv_ref[...] * jnp.dot(
        g1, w_ref[...], preferred_element_type=jnp.float32, precision=_HIGH)


def _mid(outg, s0, s1, y1, dinv, b_g1, W_g2):
    return pl.pallas_call(
        _mid_body,
        grid=(NRT,),
        in_specs=[
            pl.BlockSpec((RT, H), lambda i: (i, 0)),
            pl.BlockSpec((RT, H), lambda i: (i, 0)),
            pl.BlockSpec((RT, H), lambda i: (i, 0)),
            pl.BlockSpec((RT, H), lambda i: (i, 0)),
            pl.BlockSpec((RT, 1), lambda i: (i, 0)),
            pl.BlockSpec((1, H), lambda i: (0, 0)),
            pl.BlockSpec((H, H), lambda i: (0, 0)),
        ],
        out_specs=pl.BlockSpec((RT, H), lambda i: (i, 0)),
        out_shape=jax.ShapeDtypeStruct((NP, H), jnp.float32),
    )(outg, s0, s1, y1, dinv, b_g1, W_g2)


def _tail_body(og_ref, s0_ref, s1_ref, y2_ref, dinv_ref, b_ref, wc_ref,
               bc_ref, out_ref, acc_ref):
    i = pl.program_id(0)

    @pl.when(i == 0)
    def _init():
        acc_ref[...] = jnp.zeros((1, H), jnp.float32)

    agg = og_ref[...] + s0_ref[...] + s1_ref[...] + y2_ref[...]
    m2 = dinv_ref[...] * agg + b_ref[...]
    row = i * RT + lax.broadcasted_iota(jnp.int32, (RT, 1), 0)
    m2 = jnp.where(row < N, m2, 0.0)
    acc_ref[...] += jnp.sum(m2, axis=0, keepdims=True)

    @pl.when(i == NRT - 1)
    def _emit():
        bag = acc_ref[...] * (1.0 / N)
        out_ref[...] = jnp.dot(bag, wc_ref[...],
                               preferred_element_type=jnp.float32,
                               precision=_HIGH) + bc_ref[...]


def _tail(outg, s0, s1, y2, dinv, b_g2, Wc_pad, bc_pad):
    return pl.pallas_call(
        _tail_body,
        grid=(NRT,),
        in_specs=[
            pl.BlockSpec((RT, H), lambda i: (i, 0)),
            pl.BlockSpec((RT, H), lambda i: (i, 0)),
            pl.BlockSpec((RT, H), lambda i: (i, 0)),
            pl.BlockSpec((RT, H), lambda i: (i, 0)),
            pl.BlockSpec((RT, 1), lambda i: (i, 0)),
            pl.BlockSpec((1, H), lambda i: (0, 0)),
            pl.BlockSpec((H, H), lambda i: (0, 0)),
            pl.BlockSpec((1, H), lambda i: (0, 0)),
        ],
        out_specs=pl.BlockSpec((1, H), lambda i: (0, 0)),
        out_shape=jax.ShapeDtypeStruct((1, H), jnp.float32),
        scratch_shapes=[pltpu.VMEM((1, H), jnp.float32)],
    )(outg, s0, s1, y2, dinv, b_g2, Wc_pad, bc_pad)


def kernel(x, W_proj, b_proj, W_g1, b_g1, W_g2, b_g2, W_cls, b_cls):
    x_pad = jnp.pad(x, ((0, NP - N), (0, 0)))
    bp = b_proj.reshape(1, H)
    b1 = b_g1.reshape(1, H)
    b2 = b_g2.reshape(1, H)
    Wc_pad = jnp.zeros((H, H), jnp.float32).at[:, :2].set(W_cls)
    bc_pad = jnp.zeros((1, H), jnp.float32).at[0, :2].set(b_cls)

    h_pad, sq, sqT = _proj(x_pad, W_proj, bp)
    idx8 = _knn(h_pad, sq, sqT)
    idxk = jnp.reshape(jnp.transpose(idx8[:, 1:K + 1]), (-1,))

    parts = _indeg_sc()(idxk).reshape(_NW, NP)
    partsT = jnp.transpose(parts)

    dinv, y1 = _norm(partsT, h_pad, W_g1)
    outg1, outs1 = _mp_sc()(y1, idxk)
    y2 = _mid(outg1, outs1[0], outs1[1], y1, dinv, b1, W_g2)
    outg2, outs2 = _mp_sc()(y2, idxk)
    out128 = _tail(outg2, outs2[0], outs2[1], y2, dinv, b2, Wc_pad, bc_pad)
    return out128[:, :2]


# speedup vs baseline: 20.5719x; 1.0182x over previous
import functools

import jax
import jax.numpy as jnp
from jax import lax
from jax.experimental import pallas as pl
from jax.experimental.pallas import tpu as pltpu
from jax.experimental.pallas import tpu_sc as plsc

N = 10000
NP = 10240
D = 512
H = 128
K = 5
RT = 256
CT = 1280
NRT = NP // RT
NCT = NP // CT
INF = float("inf")
BIG = 2**30

_NC = 2
_NS = 16
_NW = _NC * _NS
_SLAB = NP // _NW
_CH = 64
_NCHK = _SLAB // _CH
_RPS = NP // _NS

_HIGH = jax.lax.Precision.HIGHEST


def _proj_body(x_ref, w_ref, b_ref, h_ref, sq_ref, sqt_ref):
    i = pl.program_id(0)
    h = jnp.dot(x_ref[...], w_ref[...], preferred_element_type=jnp.float32,
                precision=_HIGH)
    h = jnp.maximum(h + b_ref[...], 0.0)
    row = i * RT + lax.broadcasted_iota(jnp.int32, (RT, 1), 0)
    valid = row < N
    h = jnp.where(valid, h, 0.0)
    sq = jnp.sum(h * h, axis=1, keepdims=True)
    sq = jnp.where(valid, sq, 1e30)
    h_ref[...] = h
    sq_ref[...] = sq
    sqt_ref[...] = sq.reshape(1, RT)


def _proj(x_pad, W_proj, b_proj):
    return pl.pallas_call(
        _proj_body,
        grid=(NRT,),
        in_specs=[
            pl.BlockSpec((RT, D), lambda i: (i, 0)),
            pl.BlockSpec((D, H), lambda i: (0, 0)),
            pl.BlockSpec((1, H), lambda i: (0, 0)),
        ],
        out_specs=[
            pl.BlockSpec((RT, H), lambda i: (i, 0)),
            pl.BlockSpec((RT, 1), lambda i: (i, 0)),
            pl.BlockSpec((1, RT), lambda i: (0, i)),
        ],
        out_shape=[
            jax.ShapeDtypeStruct((NP, H), jnp.float32),
            jax.ShapeDtypeStruct((NP, 1), jnp.float32),
            jax.ShapeDtypeStruct((1, NP), jnp.float32),
        ],
    )(x_pad, W_proj, b_proj)


def _extract6(v, gi, width):
    pv = jnp.full((RT, 8), INF, jnp.float32)
    pi = jnp.full((RT, 8), BIG, jnp.int32)
    lane8 = lax.broadcasted_iota(jnp.int32, (RT, 8), 1)
    for p in range(K + 1):
        m = jnp.min(v, axis=1, keepdims=True)
        sel = jnp.min(jnp.where(v == m, gi, BIG), axis=1, keepdims=True)
        pv = jnp.where(lane8 == p, m, pv)
        pi = jnp.where(lane8 == p, sel, pi)
        if p < K:
            v = jnp.where(gi == sel, INF, v)
    return pv, pi


def _knn_body(hr_ref, sqr_ref, hc_ref, sqct_ref, out_ref, runv_ref, runi_ref):
    j = pl.program_id(1)

    @pl.when(j == 0)
    def _init():
        runv_ref[...] = jnp.full((RT, 8), INF, jnp.float32)
        runi_ref[...] = jnp.full((RT, 8), BIG, jnp.int32)

    d2 = (sqr_ref[...] + sqct_ref[...]
          - 2.0 * jnp.dot(hr_ref[...], hc_ref[...].T,
                          preferred_element_type=jnp.float32,
                          precision=_HIGH))
    d2 = jnp.maximum(d2, 0.0)
    gi = j * CT + lax.broadcasted_iota(jnp.int32, (RT, CT), 1)
    tv, ti = _extract6(d2, gi, CT)

    cat_v = jnp.concatenate([runv_ref[...], tv], axis=1)
    cat_i = jnp.concatenate([runi_ref[...], ti], axis=1)
    mv, mi = _extract6(cat_v, cat_i, 16)
    runv_ref[...] = mv
    runi_ref[...] = mi

    @pl.when(j == NCT - 1)
    def _emit():
        out_ref[...] = mi


def _knn(h_pad, sq, sqT):
    return pl.pallas_call(
        _knn_body,
        grid=(NRT, NCT),
        in_specs=[
            pl.BlockSpec((RT, H), lambda i, j: (i, 0)),
            pl.BlockSpec((RT, 1), lambda i, j: (i, 0)),
            pl.BlockSpec((CT, H), lambda i, j: (j, 0)),
            pl.BlockSpec((1, CT), lambda i, j: (0, j)),
        ],
        out_specs=pl.BlockSpec((RT, 8), lambda i, j: (i, 0)),
        out_shape=jax.ShapeDtypeStruct((NP, 8), jnp.int32),
        scratch_shapes=[
            pltpu.VMEM((RT, 8), jnp.float32),
            pltpu.VMEM((RT, 8), jnp.int32),
        ],
    )(h_pad, sq, h_pad, sqT)


@functools.cache
def _mesh():
    return plsc.VectorSubcoreMesh(core_axis_name="c", subcore_axis_name="s",
                                  num_cores=_NC, num_subcores=_NS)


@functools.cache
def _indeg_sc():
    return pl.kernel(
        _indeg_body,
        out_type=jax.ShapeDtypeStruct((_NW * NP,), jnp.float32),
        mesh=_mesh(),
        scratch_types=[
            pltpu.VMEM((NP,), jnp.float32),
            pltpu.VMEM((K * _SLAB,), jnp.int32),
        ],
        compiler_params=pltpu.CompilerParams(needs_layout_passes=False),
    )


def _indeg_body(idxk_hbm, out_hbm, hist, ibuf):
    c = lax.axis_index("c")
    s = lax.axis_index("s")
    wid = s * _NC + c
    base = wid * _SLAB

    for k in range(K):
        pltpu.sync_copy(idxk_hbm.at[pl.ds(k * NP + base, _SLAB)],
                        ibuf.at[pl.ds(k * _SLAB, _SLAB)])

    def _zero(i, carry):
        hist[pl.ds(i * 16, 16)] = jnp.zeros((16,), jnp.float32)
        return carry
    lax.fori_loop(0, NP // 16, _zero, 0)

    ones = jnp.ones((16,), jnp.float32)
    lane = lax.broadcasted_iota(jnp.int32, (16,), 0)
    for k in range(K):
        def _acc(i, carry, k=k):
            iv = ibuf[pl.ds(k * _SLAB + i * 16, 16)]
            src = base + i * 16 + lane
            iv = jnp.where(src < N, iv, N)
            plsc.addupdate_scatter(hist, [iv], ones)
            return carry
        lax.fori_loop(0, _SLAB // 16, _acc, 0)

    pltpu.sync_copy(hist, out_hbm.at[pl.ds(wid * NP, NP)])


_HALF = NP // 2
_HPS = _HALF // _NS


@functools.cache
def _mp_sc():
    return pl.kernel(
        _mp_body,
        out_type=(
            jax.ShapeDtypeStruct((NP, H), jnp.float32),
            jax.ShapeDtypeStruct((_NC, NP, H), jnp.float32),
        ),
        mesh=_mesh(),
        scratch_types=[
            pltpu.VMEM((K * _SLAB,), jnp.int32),
            pltpu.VMEM((_CH,), jnp.int32),
            pltpu.VMEM((K, _CH, H), jnp.float32),
            pltpu.VMEM((_CH, H), jnp.float32),
            pltpu.VMEM((_CH, H), jnp.float32),
            pltpu.VMEM_SHARED((_HALF + 8, H), jnp.float32),
            pltpu.SemaphoreType.DMA,
            pltpu.SemaphoreType.DMA,
        ],
        compiler_params=pltpu.CompilerParams(needs_layout_passes=False),
    )


def _mp_body(y_hbm, idxk_hbm, outg_hbm, outs_hbm, ibuf, ibw, gb, yb, zb,
             spacc, sem, sem2):
    c = lax.axis_index("c")
    s = lax.axis_index("s")
    wid = s * _NC + c
    base = wid * _SLAB
    nq = _CH // 16

    zeros16 = jnp.zeros((16,), jnp.float32)

    def _zrow(r, carry):
        for l in range(H // 16):
            zb[r, pl.ds(l * 16, 16)] = zeros16
        return carry
    lax.fori_loop(0, _CH, _zrow, 0)

    for k in range(K):
        pltpu.sync_copy(idxk_hbm.at[pl.ds(k * NP + base, _SLAB)],
                        ibuf.at[pl.ds(k * _SLAB, _SLAB)])

    def _gchunk(ch, carry):
        cb = base + ch * _CH
        cps = []
        for k in range(K):
            cps.append(pltpu.async_copy(
                y_hbm.at[ibuf.at[pl.ds(k * _SLAB + ch * _CH, _CH)]],
                gb.at[k], sem))
        for cp in cps:
            cp.wait()

        def _sum(r, c2):
            for l in range(H // 16):
                sl = pl.ds(l * 16, 16)
                acc = gb[0, r, sl]
                for k in range(1, K):
                    acc = acc + gb[k, r, sl]
                gb[0, r, sl] = acc
            return c2
        lax.fori_loop(0, _CH, _sum, 0)
        pltpu.sync_copy(gb.at[0], outg_hbm.at[pl.ds(cb, _CH)])
        return carry
    lax.fori_loop(0, _NCHK, _gchunk, 0)

    for half in range(2):
        lo = half * _HALF

        for t in range(_HPS // _CH):
            pltpu.sync_copy(zb, spacc.at[pl.ds(s * _HPS + t * _CH, _CH)])
        plsc.subcore_barrier()

        def _schunk(ch, carry, lo=lo):
            cb = base + ch * _CH
            pltpu.sync_copy(y_hbm.at[pl.ds(cb, _CH)], yb)
            for k in range(K):
                for q in range(nq):
                    iv = ibuf[pl.ds(k * _SLAB + ch * _CH + q * 16, 16)]
                    lv = iv - lo
                    ok = jnp.logical_and(lv >= 0, lv < _HALF)
                    ibw[pl.ds(q * 16, 16)] = jnp.where(ok, lv, _HALF)
                pltpu.sync_copy(yb, spacc.at[ibw], add=True)
            return carry
        lax.fori_loop(0, _NCHK, _schunk, 0)

        plsc.subcore_barrier()
        pltpu.sync_copy(spacc.at[pl.ds(s * _HPS, _HPS)],
                        outs_hbm.at[c, pl.ds(lo + s * _HPS, _HPS)])
        plsc.subcore_barrier()


def _norm_body(pt_ref, h_ref, w_ref, dinv_ref, y_ref):
    deg = (K + 1.0) + jnp.sum(pt_ref[...], axis=1, keepdims=True)
    dinv = lax.rsqrt(deg)
    y = dinv * jnp.dot(h_ref[...], w_ref[...],
                       preferred_element_type=jnp.float32, precision=_HIGH)
    dinv_ref[...] = dinv
    y_ref[...] = y


def _norm(partsT, h_pad, W):
    return pl.pallas_call(
        _norm_body,
        grid=(NRT,),
        in_specs=[
            pl.BlockSpec((RT, _NW), lambda i: (i, 0)),
            pl.BlockSpec((RT, H), lambda i: (i, 0)),
            pl.BlockSpec((H, H), lambda i: (0, 0)),
        ],
        out_specs=[
            pl.BlockSpec((RT, 1), lambda i: (i, 0)),
            pl.BlockSpec((RT, H), lambda i: (i, 0)),
        ],
        out_shape=[
            jax.ShapeDtypeStruct((NP, 1), jnp.float32),
            jax.ShapeDtypeStruct((NP, H), jnp.float32),
        ],
    )(partsT, h_pad, W)


def _mid_body(og_ref, s0_ref, s1_ref, y1_ref, dinv_ref, b_ref, w_ref, y2_ref):
    i = pl.program_id(0)
    agg = og_ref[...] + s0_ref[...] + s1_ref[...] + y1_ref[...]
    g1 = jnp.maximum(dinv_ref[...] * agg + b_ref[...], 0.0)
    row = i * RT + lax.broadcasted_iota(jnp.int32, (RT, 1), 0)
    g1 = jnp.where(row < N, g1, 0.0)
    y2_ref[...] = dinv_ref[...] * jnp.dot(
        g1, w_ref[...], preferred_element_type=jnp.float32, precision=_HIGH)


def _mid(outg, s0, s1, y1, dinv, b_g1, W_g2):
    return pl.pallas_call(
        _mid_body,
        grid=(NRT,),
        in_specs=[
            pl.BlockSpec((RT, H), lambda i: (i, 0)),
            pl.BlockSpec((RT, H), lambda i: (i, 0)),
            pl.BlockSpec((RT, H), lambda i: (i, 0)),
            pl.BlockSpec((RT, H), lambda i: (i, 0)),
            pl.BlockSpec((RT, 1), lambda i: (i, 0)),
            pl.BlockSpec((1, H), lambda i: (0, 0)),
            pl.BlockSpec((H, H), lambda i: (0, 0)),
        ],
        out_specs=pl.BlockSpec((RT, H), lambda i: (i, 0)),
        out_shape=jax.ShapeDtypeStruct((NP, H), jnp.float32),
    )(outg, s0, s1, y1, dinv, b_g1, W_g2)


def _tail_body(og_ref, s0_ref, s1_ref, y2_ref, dinv_ref, b_ref, wc_ref,
               bc_ref, out_ref, acc_ref):
    i = pl.program_id(0)

    @pl.when(i == 0)
    def _init():
        acc_ref[...] = jnp.zeros((1, H), jnp.float32)

    agg = og_ref[...] + s0_ref[...] + s1_ref[...] + y2_ref[...]
    m2 = dinv_ref[...] * agg + b_ref[...]
    row = i * RT + lax.broadcasted_iota(jnp.int32, (RT, 1), 0)
    m2 = jnp.where(row < N, m2, 0.0)
    acc_ref[...] += jnp.sum(m2, axis=0, keepdims=True)

    @pl.when(i == NRT - 1)
    def _emit():
        bag = acc_ref[...] * (1.0 / N)
        out_ref[...] = jnp.dot(bag, wc_ref[...],
                               preferred_element_type=jnp.float32,
                               precision=_HIGH) + bc_ref[...]


def _tail(outg, s0, s1, y2, dinv, b_g2, Wc_pad, bc_pad):
    return pl.pallas_call(
        _tail_body,
        grid=(NRT,),
        in_specs=[
            pl.BlockSpec((RT, H), lambda i: (i, 0)),
            pl.BlockSpec((RT, H), lambda i: (i, 0)),
            pl.BlockSpec((RT, H), lambda i: (i, 0)),
            pl.BlockSpec((RT, H), lambda i: (i, 0)),
            pl.BlockSpec((RT, 1), lambda i: (i, 0)),
            pl.BlockSpec((1, H), lambda i: (0, 0)),
            pl.BlockSpec((H, H), lambda i: (0, 0)),
            pl.BlockSpec((1, H), lambda i: (0, 0)),
        ],
        out_specs=pl.BlockSpec((1, H), lambda i: (0, 0)),
        out_shape=jax.ShapeDtypeStruct((1, H), jnp.float32),
        scratch_shapes=[pltpu.VMEM((1, H), jnp.float32)],
    )(outg, s0, s1, y2, dinv, b_g2, Wc_pad, bc_pad)


def kernel(x, W_proj, b_proj, W_g1, b_g1, W_g2, b_g2, W_cls, b_cls):
    x_pad = jnp.pad(x, ((0, NP - N), (0, 0)))
    bp = b_proj.reshape(1, H)
    b1 = b_g1.reshape(1, H)
    b2 = b_g2.reshape(1, H)
    Wc_pad = jnp.zeros((H, H), jnp.float32).at[:, :2].set(W_cls)
    bc_pad = jnp.zeros((1, H), jnp.float32).at[0, :2].set(b_cls)

    h_pad, sq, sqT = _proj(x_pad, W_proj, bp)
    idx8 = _knn(h_pad, sq, sqT)
    idxk = jnp.reshape(jnp.transpose(idx8[:, 1:K + 1]), (-1,))

    parts = _indeg_sc()(idxk).reshape(_NW, NP)
    partsT = jnp.transpose(parts)

    dinv, y1 = _norm(partsT, h_pad, W_g1)
    outg1, outs1 = _mp_sc()(y1, idxk)
    y2 = _mid(outg1, outs1[0], outs1[1], y1, dinv, b1, W_g2)
    outg2, outs2 = _mp_sc()(y2, idxk)
    out128 = _tail(outg2, outs2[0], outs2[1], y2, dinv, b2, Wc_pad, bc_pad)
    return out128[:, :2]


# speedup vs baseline: 23.8162x; 1.1577x over previous
import functools

import jax
import jax.numpy as jnp
from jax import lax
from jax.experimental import pallas as pl
from jax.experimental.pallas import tpu as pltpu
from jax.experimental.pallas import tpu_sc as plsc

N = 10000
NP = 10240
D = 512
H = 128
K = 5
RT = 256
CT = 1280
NRT = NP // RT
NCT = NP // CT
INF = float("inf")
BIG = 2**30
BIGF = 1e30

_NC = 2
_NS = 16
_NW = _NC * _NS
_SLAB = NP // _NW
_CH = 64
_NCHK = _SLAB // _CH
_RPS = NP // _NS

_HIGH = jax.lax.Precision.HIGHEST


def _proj_body(x_ref, w_ref, b_ref, h_ref, sq_ref, sqt_ref):
    i = pl.program_id(0)
    h = jnp.dot(x_ref[...], w_ref[...], preferred_element_type=jnp.float32,
                precision=_HIGH)
    h = jnp.maximum(h + b_ref[...], 0.0)
    row = i * RT + lax.broadcasted_iota(jnp.int32, (RT, 1), 0)
    valid = row < N
    h = jnp.where(valid, h, 0.0)
    sq = jnp.sum(h * h, axis=1, keepdims=True)
    sq = jnp.where(valid, sq, 1e30)
    h_ref[...] = h
    sq_ref[...] = sq
    sqt_ref[...] = sq.reshape(1, RT)


def _proj(x_pad, W_proj, b_proj):
    return pl.pallas_call(
        _proj_body,
        grid=(NRT,),
        in_specs=[
            pl.BlockSpec((RT, D), lambda i: (i, 0)),
            pl.BlockSpec((D, H), lambda i: (0, 0)),
            pl.BlockSpec((1, H), lambda i: (0, 0)),
        ],
        out_specs=[
            pl.BlockSpec((RT, H), lambda i: (i, 0)),
            pl.BlockSpec((RT, 1), lambda i: (i, 0)),
            pl.BlockSpec((1, RT), lambda i: (0, i)),
        ],
        out_shape=[
            jax.ShapeDtypeStruct((NP, H), jnp.float32),
            jax.ShapeDtypeStruct((NP, 1), jnp.float32),
            jax.ShapeDtypeStruct((1, NP), jnp.float32),
        ],
    )(x_pad, W_proj, b_proj)


def _extract6(v, gi, width):
    pv = jnp.full((RT, 8), INF, jnp.float32)
    pi = jnp.full((RT, 8), BIGF, jnp.float32)
    lane8 = lax.broadcasted_iota(jnp.int32, (RT, 8), 1)
    for p in range(K + 1):
        m = jnp.min(v, axis=1, keepdims=True)
        sel = jnp.min(jnp.where(v == m, gi, BIGF), axis=1, keepdims=True)
        pv = jnp.where(lane8 == p, m, pv)
        pi = jnp.where(lane8 == p, sel, pi)
        if p < K:
            v = jnp.where(gi == sel, INF, v)
    return pv, pi


def _knn_body(hr_ref, sqr_ref, hc_ref, sqct_ref, out_ref, runv_ref, runi_ref):
    j = pl.program_id(1)

    @pl.when(j == 0)
    def _init():
        runv_ref[...] = jnp.full((RT, 8), INF, jnp.float32)
        runi_ref[...] = jnp.full((RT, 8), BIGF, jnp.float32)

    d2 = (sqr_ref[...] + sqct_ref[...]
          - 2.0 * jnp.dot(hr_ref[...], hc_ref[...].T,
                          preferred_element_type=jnp.float32,
                          precision=_HIGH))
    dist = jnp.sqrt(jnp.maximum(d2, 0.0))
    gi = (j * CT + lax.broadcasted_iota(jnp.int32, (RT, CT), 1)
          ).astype(jnp.float32)
    tv, ti = _extract6(dist, gi, CT)

    cat_v = jnp.concatenate([runv_ref[...], tv], axis=1)
    cat_i = jnp.concatenate([runi_ref[...], ti], axis=1)
    mv, mi = _extract6(cat_v, cat_i, 16)
    runv_ref[...] = mv
    runi_ref[...] = mi

    @pl.when(j == NCT - 1)
    def _emit():
        out_ref[...] = mi.astype(jnp.int32)


def _knn(h_pad, sq, sqT):
    return pl.pallas_call(
        _knn_body,
        grid=(NRT, NCT),
        in_specs=[
            pl.BlockSpec((RT, H), lambda i, j: (i, 0)),
            pl.BlockSpec((RT, 1), lambda i, j: (i, 0)),
            pl.BlockSpec((CT, H), lambda i, j: (j, 0)),
            pl.BlockSpec((1, CT), lambda i, j: (0, j)),
        ],
        out_specs=pl.BlockSpec((RT, 8), lambda i, j: (i, 0)),
        out_shape=jax.ShapeDtypeStruct((NP, 8), jnp.int32),
        scratch_shapes=[
            pltpu.VMEM((RT, 8), jnp.float32),
            pltpu.VMEM((RT, 8), jnp.float32),
        ],
    )(h_pad, sq, h_pad, sqT)


@functools.cache
def _mesh():
    return plsc.VectorSubcoreMesh(core_axis_name="c", subcore_axis_name="s",
                                  num_cores=_NC, num_subcores=_NS)


@functools.cache
def _indeg_sc():
    return pl.kernel(
        _indeg_body,
        out_type=jax.ShapeDtypeStruct((_NW * NP,), jnp.float32),
        mesh=_mesh(),
        scratch_types=[
            pltpu.VMEM((NP,), jnp.float32),
            pltpu.VMEM((K * _SLAB,), jnp.int32),
        ],
        compiler_params=pltpu.CompilerParams(needs_layout_passes=False),
    )


def _indeg_body(idxk_hbm, out_hbm, hist, ibuf):
    c = lax.axis_index("c")
    s = lax.axis_index("s")
    wid = s * _NC + c
    base = wid * _SLAB

    for k in range(K):
        pltpu.sync_copy(idxk_hbm.at[pl.ds(k * NP + base, _SLAB)],
                        ibuf.at[pl.ds(k * _SLAB, _SLAB)])

    def _zero(i, carry):
        hist[pl.ds(i * 16, 16)] = jnp.zeros((16,), jnp.float32)
        return carry
    lax.fori_loop(0, NP // 16, _zero, 0)

    ones = jnp.ones((16,), jnp.float32)
    lane = lax.broadcasted_iota(jnp.int32, (16,), 0)
    for k in range(K):
        def _acc(i, carry, k=k):
            iv = ibuf[pl.ds(k * _SLAB + i * 16, 16)]
            src = base + i * 16 + lane
            iv = jnp.where(src < N, iv, N)
            plsc.addupdate_scatter(hist, [iv], ones)
            return carry
        lax.fori_loop(0, _SLAB // 16, _acc, 0)

    pltpu.sync_copy(hist, out_hbm.at[pl.ds(wid * NP, NP)])


_HALF = NP // 2
_HPS = _HALF // _NS


@functools.cache
def _mp_sc():
    return pl.kernel(
        _mp_body,
        out_type=(
            jax.ShapeDtypeStruct((NP, H), jnp.float32),
            jax.ShapeDtypeStruct((_NC, NP, H), jnp.float32),
        ),
        mesh=_mesh(),
        scratch_types=[
            pltpu.VMEM((K * _SLAB,), jnp.int32),
            pltpu.VMEM((_CH,), jnp.int32),
            pltpu.VMEM((K, _CH, H), jnp.float32),
            pltpu.VMEM((_CH, H), jnp.float32),
            pltpu.VMEM((_CH, H), jnp.float32),
            pltpu.VMEM_SHARED((_HALF + 8, H), jnp.float32),
            pltpu.SemaphoreType.DMA,
            pltpu.SemaphoreType.DMA,
        ],
        compiler_params=pltpu.CompilerParams(needs_layout_passes=False),
    )


def _mp_body(y_hbm, idxk_hbm, outg_hbm, outs_hbm, ibuf, ibw, gb, yb, zb,
             spacc, sem, sem2):
    c = lax.axis_index("c")
    s = lax.axis_index("s")
    wid = s * _NC + c
    base = wid * _SLAB
    nq = _CH // 16

    zeros16 = jnp.zeros((16,), jnp.float32)

    def _zrow(r, carry):
        for l in range(H // 16):
            zb[r, pl.ds(l * 16, 16)] = zeros16
        return carry
    lax.fori_loop(0, _CH, _zrow, 0)

    for k in range(K):
        pltpu.sync_copy(idxk_hbm.at[pl.ds(k * NP + base, _SLAB)],
                        ibuf.at[pl.ds(k * _SLAB, _SLAB)])

    def _gchunk(ch, carry):
        cb = base + ch * _CH
        cps = []
        for k in range(K):
            cps.append(pltpu.async_copy(
                y_hbm.at[ibuf.at[pl.ds(k * _SLAB + ch * _CH, _CH)]],
                gb.at[k], sem))
        for cp in cps:
            cp.wait()

        def _sum(r, c2):
            for l in range(H // 16):
                sl = pl.ds(l * 16, 16)
                acc = gb[0, r, sl]
                for k in range(1, K):
                    acc = acc + gb[k, r, sl]
                gb[0, r, sl] = acc
            return c2
        lax.fori_loop(0, _CH, _sum, 0)
        pltpu.sync_copy(gb.at[0], outg_hbm.at[pl.ds(cb, _CH)])
        return carry
    lax.fori_loop(0, _NCHK, _gchunk, 0)

    for half in range(2):
        lo = half * _HALF

        for t in range(_HPS // _CH):
            pltpu.sync_copy(zb, spacc.at[pl.ds(s * _HPS + t * _CH, _CH)])
        plsc.subcore_barrier()

        def _schunk(ch, carry, lo=lo):
            cb = base + ch * _CH
            pltpu.sync_copy(y_hbm.at[pl.ds(cb, _CH)], yb)
            for k in range(K):
                for q in range(nq):
                    iv = ibuf[pl.ds(k * _SLAB + ch * _CH + q * 16, 16)]
                    lv = iv - lo
                    ok = jnp.logical_and(lv >= 0, lv < _HALF)
                    ibw[pl.ds(q * 16, 16)] = jnp.where(ok, lv, _HALF)
                pltpu.sync_copy(yb, spacc.at[ibw], add=True)
            return carry
        lax.fori_loop(0, _NCHK, _schunk, 0)

        plsc.subcore_barrier()
        pltpu.sync_copy(spacc.at[pl.ds(s * _HPS, _HPS)],
                        outs_hbm.at[c, pl.ds(lo + s * _HPS, _HPS)])
        plsc.subcore_barrier()


def _norm_body(pt_ref, h_ref, w_ref, dinv_ref, y_ref):
    deg = (K + 1.0) + jnp.sum(pt_ref[...], axis=1, keepdims=True)
    dinv = lax.rsqrt(deg)
    y = dinv * jnp.dot(h_ref[...], w_ref[...],
                       preferred_element_type=jnp.float32, precision=_HIGH)
    dinv_ref[...] = dinv
    y_ref[...] = y


def _norm(partsT, h_pad, W):
    return pl.pallas_call(
        _norm_body,
        grid=(NRT,),
        in_specs=[
            pl.BlockSpec((RT, _NW), lambda i: (i, 0)),
            pl.BlockSpec((RT, H), lambda i: (i, 0)),
            pl.BlockSpec((H, H), lambda i: (0, 0)),
        ],
        out_specs=[
            pl.BlockSpec((RT, 1), lambda i: (i, 0)),
            pl.BlockSpec((RT, H), lambda i: (i, 0)),
        ],
        out_shape=[
            jax.ShapeDtypeStruct((NP, 1), jnp.float32),
            jax.ShapeDtypeStruct((NP, H), jnp.float32),
        ],
    )(partsT, h_pad, W)


def _mid_body(og_ref, s0_ref, s1_ref, y1_ref, dinv_ref, b_ref, w_ref, y2_ref):
    i = pl.program_id(0)
    agg = og_ref[...] + s0_ref[...] + s1_ref[...] + y1_ref[...]
    g1 = jnp.maximum(dinv_ref[...] * agg + b_ref[...], 0.0)
    row = i * RT + lax.broadcasted_iota(jnp.int32, (RT, 1), 0)
    g1 = jnp.where(row < N, g1, 0.0)
    y2_ref[...] = dinv_ref[...] * jnp.dot(
        g1, w_ref[...], preferred_element_type=jnp.float32, precision=_HIGH)


def _mid(outg, s0, s1, y1, dinv, b_g1, W_g2):
    return pl.pallas_call(
        _mid_body,
        grid=(NRT,),
        in_specs=[
            pl.BlockSpec((RT, H), lambda i: (i, 0)),
            pl.BlockSpec((RT, H), lambda i: (i, 0)),
            pl.BlockSpec((RT, H), lambda i: (i, 0)),
            pl.BlockSpec((RT, H), lambda i: (i, 0)),
            pl.BlockSpec((RT, 1), lambda i: (i, 0)),
            pl.BlockSpec((1, H), lambda i: (0, 0)),
            pl.BlockSpec((H, H), lambda i: (0, 0)),
        ],
        out_specs=pl.BlockSpec((RT, H), lambda i: (i, 0)),
        out_shape=jax.ShapeDtypeStruct((NP, H), jnp.float32),
    )(outg, s0, s1, y1, dinv, b_g1, W_g2)


def _tail_body(og_ref, s0_ref, s1_ref, y2_ref, dinv_ref, b_ref, wc_ref,
               bc_ref, out_ref, acc_ref):
    i = pl.program_id(0)

    @pl.when(i == 0)
    def _init():
        acc_ref[...] = jnp.zeros((1, H), jnp.float32)

    agg = og_ref[...] + s0_ref[...] + s1_ref[...] + y2_ref[...]
    m2 = dinv_ref[...] * agg + b_ref[...]
    row = i * RT + lax.broadcasted_iota(jnp.int32, (RT, 1), 0)
    m2 = jnp.where(row < N, m2, 0.0)
    acc_ref[...] += jnp.sum(m2, axis=0, keepdims=True)

    @pl.when(i == NRT - 1)
    def _emit():
        bag = acc_ref[...] * (1.0 / N)
        out_ref[...] = jnp.dot(bag, wc_ref[...],
                               preferred_element_type=jnp.float32,
                               precision=_HIGH) + bc_ref[...]


def _tail(outg, s0, s1, y2, dinv, b_g2, Wc_pad, bc_pad):
    return pl.pallas_call(
        _tail_body,
        grid=(NRT,),
        in_specs=[
            pl.BlockSpec((RT, H), lambda i: (i, 0)),
            pl.BlockSpec((RT, H), lambda i: (i, 0)),
            pl.BlockSpec((RT, H), lambda i: (i, 0)),
            pl.BlockSpec((RT, H), lambda i: (i, 0)),
            pl.BlockSpec((RT, 1), lambda i: (i, 0)),
            pl.BlockSpec((1, H), lambda i: (0, 0)),
            pl.BlockSpec((H, H), lambda i: (0, 0)),
            pl.BlockSpec((1, H), lambda i: (0, 0)),
        ],
        out_specs=pl.BlockSpec((1, H), lambda i: (0, 0)),
        out_shape=jax.ShapeDtypeStruct((1, H), jnp.float32),
        scratch_shapes=[pltpu.VMEM((1, H), jnp.float32)],
    )(outg, s0, s1, y2, dinv, b_g2, Wc_pad, bc_pad)


def kernel(x, W_proj, b_proj, W_g1, b_g1, W_g2, b_g2, W_cls, b_cls):
    x_pad = jnp.pad(x, ((0, NP - N), (0, 0)))
    bp = b_proj.reshape(1, H)
    b1 = b_g1.reshape(1, H)
    b2 = b_g2.reshape(1, H)
    Wc_pad = jnp.zeros((H, H), jnp.float32).at[:, :2].set(W_cls)
    bc_pad = jnp.zeros((1, H), jnp.float32).at[0, :2].set(b_cls)

    h_pad, sq, sqT = _proj(x_pad, W_proj, bp)
    idx8 = _knn(h_pad, sq, sqT)
    idxk = jnp.reshape(jnp.transpose(idx8[:, 1:K + 1]), (-1,))

    parts = _indeg_sc()(idxk).reshape(_NW, NP)
    partsT = jnp.transpose(parts)

    dinv, y1 = _norm(partsT, h_pad, W_g1)
    outg1, outs1 = _mp_sc()(y1, idxk)
    y2 = _mid(outg1, outs1[0], outs1[1], y1, dinv, b1, W_g2)
    outg2, outs2 = _mp_sc()(y2, idxk)
    out128 = _tail(outg2, outs2[0], outs2[1], y2, dinv, b2, Wc_pad, bc_pad)
    return out128[:, :2]
